# baseline (speedup 1.0000x reference)
import functools

import jax
import jax.numpy as jnp
from jax.experimental import pallas as pl
from jax.experimental.pallas import tpu as pltpu

F = 100
M = 20
H1 = 64
H2 = 32
PAIR_LANES = 2 * M


def _body(x_ref, Wint_ref, W1a_ref, W1b_ref, b1_ref, W2T_ref, b2_ref, W3v_ref,
          b3_ref, Wc1T_ref, bc1_ref, Wc2T_ref, bc2_ref,
          feat_ref, vals_ref, sel_ref, csum_ref, S_ref, *, num_tiles, inv_b):
    pid = pl.program_id(0)

    @pl.when(pid == 0)
    def _topk():
        W = Wint_ref[...]
        row = jax.lax.broadcasted_iota(jnp.int32, (F, F), 0)
        col = jax.lax.broadcasted_iota(jnp.int32, (F, F), 1)
        flat = row * F + col
        s = jnp.where(col > row, jax.nn.sigmoid(W), -1.0)
        lane = jax.lax.broadcasted_iota(jnp.int32, (1, M), 1)
        vals = jnp.zeros((1, M), jnp.float32)
        idxs = jnp.zeros((1, M), jnp.int32)
        for k in range(M):
            m = jnp.max(s)
            cand = jnp.where(s == m, flat, jnp.int32(2**31 - 1))
            idx = jnp.min(cand)
            vals = jnp.where(lane == k, m, vals)
            idxs = jnp.where(lane == k, idx, idxs)
            s = jnp.where(flat == idx, -1.0, s)
        sel_i = idxs // F
        sel_j = idxs - sel_i * F
        sel = jnp.concatenate([sel_i, sel_j], axis=1)
        vals_ref[...] = vals
        sel_ref[...] = sel
        frow = jax.lax.broadcasted_iota(jnp.int32, (F, PAIR_LANES), 0)
        S_ref[...] = (frow == jnp.broadcast_to(sel, (F, PAIR_LANES))).astype(
            jnp.float32)

    xt = x_ref[...]
    pairs = jnp.dot(xt, S_ref[...], preferred_element_type=jnp.float32)
    xs = pairs[:, :M]
    xj = pairs[:, M:]
    hc = jnp.maximum(
        jnp.dot(xt, Wc1T_ref[...], preferred_element_type=jnp.float32)
        + bc1_ref[...], 0.0)
    cw = jax.nn.sigmoid(
        jnp.dot(hc, Wc2T_ref[...], preferred_element_type=jnp.float32)
        + bc2_ref[...])
    cols = []
    for m in range(M):
        h1 = jnp.maximum(
            xs[:, m:m + 1] * W1a_ref[m:m + 1, :]
            + xj[:, m:m + 1] * W1b_ref[m:m + 1, :] + b1_ref[m:m + 1, :], 0.0)
        h2 = jnp.maximum(
            jnp.dot(h1, W2T_ref[m], preferred_element_type=jnp.float32)
            + b2_ref[m:m + 1, :], 0.0)
        cols.append(jnp.sum(h2 * W3v_ref[m:m + 1, :], axis=1, keepdims=True))
    ocols = jnp.concatenate(cols, axis=1)
    feat_ref[...] = (ocols + b3_ref[...]) * cw

    @pl.when(pid == 0)
    def _init():
        csum_ref[...] = jnp.zeros_like(csum_ref)

    csum_ref[...] += jnp.sum(cw, axis=0, keepdims=True)

    @pl.when(pid == num_tiles - 1)
    def _fin():
        csum_ref[...] *= inv_b


@jax.jit
def kernel(x, W_int, W1, b1, W2, b2, W3, b3, Wc1, bc1, Wc2, bc2):
    B = x.shape[0]
    T = 1024
    n = B // T
    W1a = W1[:, :, 0]
    W1b = W1[:, :, 1]
    W2T = jnp.transpose(W2, (0, 2, 1))
    W3v = W3[:, 0, :]
    b3r = jnp.reshape(b3, (1, M))
    Wc1T = Wc1.T
    Wc2T = Wc2.T
    bc1r = bc1.reshape(1, H1)
    bc2r = bc2.reshape(1, M)

    feat, vals, sel, cmean = pl.pallas_call(
        functools.partial(_body, num_tiles=n, inv_b=1.0 / B),
        grid=(n,),
        in_specs=[
            pl.BlockSpec((T, F), lambda i: (i, 0)),
            pl.BlockSpec((F, F), lambda i: (0, 0)),
            pl.BlockSpec((M, H1), lambda i: (0, 0)),
            pl.BlockSpec((M, H1), lambda i: (0, 0)),
            pl.BlockSpec((M, H1), lambda i: (0, 0)),
            pl.BlockSpec((M, H1, H2), lambda i: (0, 0, 0)),
            pl.BlockSpec((M, H2), lambda i: (0, 0)),
            pl.BlockSpec((M, H2), lambda i: (0, 0)),
            pl.BlockSpec((1, M), lambda i: (0, 0)),
            pl.BlockSpec((F, H1), lambda i: (0, 0)),
            pl.BlockSpec((1, H1), lambda i: (0, 0)),
            pl.BlockSpec((H1, M), lambda i: (0, 0)),
            pl.BlockSpec((1, M), lambda i: (0, 0)),
        ],
        out_specs=[
            pl.BlockSpec((T, M), lambda i: (i, 0)),
            pl.BlockSpec((1, M), lambda i: (0, 0)),
            pl.BlockSpec((1, PAIR_LANES), lambda i: (0, 0)),
            pl.BlockSpec((1, M), lambda i: (0, 0)),
        ],
        out_shape=[
            jax.ShapeDtypeStruct((B, M), jnp.float32),
            jax.ShapeDtypeStruct((1, M), jnp.float32),
            jax.ShapeDtypeStruct((1, PAIR_LANES), jnp.int32),
            jax.ShapeDtypeStruct((1, M), jnp.float32),
        ],
        scratch_shapes=[pltpu.VMEM((F, PAIR_LANES), jnp.float32)],
    )(x, W_int, W1a, W1b, b1, W2T, b2, W3v, b3r, Wc1T, bc1r, Wc2T, bc2r)
    selected_pairs = jnp.stack([sel[0, :M], sel[0, M:]], axis=1)
    return (feat, vals[0], cmean[0], selected_pairs)


# speedup vs baseline: 1.8090x; 1.8090x over previous
import functools

import jax
import jax.numpy as jnp
from jax.experimental import pallas as pl
from jax.experimental.pallas import tpu as pltpu

F = 100
M = 20
H1 = 64
H2 = 32
PAIR_LANES = 2 * M
GRP = 4
NG = M // GRP
H1F = M * H1
H2F = M * H2


def _body(x_ref, Wint_ref, W1row_ref, b1f_ref, W2bd_ref, b2f_ref, W3col_ref,
          b3_ref, Wc1T_ref, bc1_ref, Wc2T_ref, bc2_ref,
          feat_ref, vals_ref, sel_ref, csum_ref, G_ref, *, num_tiles, inv_b):
    pid = pl.program_id(0)

    @pl.when(pid == 0)
    def _topk():
        W = Wint_ref[...]
        row = jax.lax.broadcasted_iota(jnp.int32, (F, F), 0)
        col = jax.lax.broadcasted_iota(jnp.int32, (F, F), 1)
        flat = row * F + col
        s = jnp.where(col > row, jax.nn.sigmoid(W), -1.0)
        lane = jax.lax.broadcasted_iota(jnp.int32, (1, M), 1)
        vals = jnp.zeros((1, M), jnp.float32)
        idxs = jnp.zeros((1, M), jnp.int32)
        for k in range(M):
            m = jnp.max(s)
            cand = jnp.where(s == m, flat, jnp.int32(2**31 - 1))
            idx = jnp.min(cand)
            vals = jnp.where(lane == k, m, vals)
            idxs = jnp.where(lane == k, idx, idxs)
            s = jnp.where(flat == idx, -1.0, s)
        sel_i = idxs // F
        sel_j = idxs - sel_i * F
        sel = jnp.concatenate([sel_i, sel_j], axis=1)
        vals_ref[...] = vals
        sel_ref[...] = sel
        frow = jax.lax.broadcasted_iota(jnp.int32, (F, PAIR_LANES), 0)
        S = (frow == jnp.broadcast_to(sel, (F, PAIR_LANES))).astype(
            jnp.float32)
        G_ref[...] = jnp.dot(S, W1row_ref[...],
                             preferred_element_type=jnp.float32)

    xt = x_ref[...]
    hc = jnp.maximum(
        jnp.dot(xt, Wc1T_ref[...], preferred_element_type=jnp.float32)
        + bc1_ref[...], 0.0)
    cw = jax.nn.sigmoid(
        jnp.dot(hc, Wc2T_ref[...], preferred_element_type=jnp.float32)
        + bc2_ref[...])

    h1 = jnp.maximum(
        jnp.dot(xt, G_ref[...], preferred_element_type=jnp.float32)
        + b1f_ref[...], 0.0)
    h2g = []
    for g in range(NG):
        hg = jnp.dot(h1[:, g * GRP * H1:(g + 1) * GRP * H1], W2bd_ref[g],
                     preferred_element_type=jnp.float32)
        h2g.append(jnp.maximum(
            hg + b2f_ref[:, g * GRP * H2:(g + 1) * GRP * H2], 0.0))
    h2 = jnp.concatenate(h2g, axis=1)
    o = jnp.dot(h2, W3col_ref[...], preferred_element_type=jnp.float32)
    feat_ref[...] = (o + b3_ref[...]) * cw

    @pl.when(pid == 0)
    def _init():
        csum_ref[...] = jnp.zeros_like(csum_ref)

    csum_ref[...] += jnp.sum(cw, axis=0, keepdims=True)

    @pl.when(pid == num_tiles - 1)
    def _fin():
        csum_ref[...] *= inv_b


@jax.jit
def kernel(x, W_int, W1, b1, W2, b2, W3, b3, Wc1, bc1, Wc2, bc2):
    B = x.shape[0]
    T = 1024
    n = B // T
    W1row = jnp.zeros((PAIR_LANES, H1F), jnp.float32)
    for m in range(M):
        W1row = W1row.at[m, m * H1:(m + 1) * H1].set(W1[m, :, 0])
        W1row = W1row.at[M + m, m * H1:(m + 1) * H1].set(W1[m, :, 1])
    b1f = b1.reshape(1, H1F)
    W2T = jnp.transpose(W2, (0, 2, 1))
    W2bd = jnp.zeros((NG, GRP * H1, GRP * H2), jnp.float32)
    for g in range(NG):
        for ml in range(GRP):
            W2bd = W2bd.at[g, ml * H1:(ml + 1) * H1,
                           ml * H2:(ml + 1) * H2].set(W2T[g * GRP + ml])
    b2f = b2.reshape(1, H2F)
    W3col = jnp.zeros((H2F, M), jnp.float32)
    for m in range(M):
        W3col = W3col.at[m * H2:(m + 1) * H2, m].set(W3[m, 0, :])
    b3r = jnp.reshape(b3, (1, M))
    Wc1T = Wc1.T
    Wc2T = Wc2.T
    bc1r = bc1.reshape(1, H1)
    bc2r = bc2.reshape(1, M)

    feat, vals, sel, cmean = pl.pallas_call(
        functools.partial(_body, num_tiles=n, inv_b=1.0 / B),
        grid=(n,),
        in_specs=[
            pl.BlockSpec((T, F), lambda i: (i, 0)),
            pl.BlockSpec((F, F), lambda i: (0, 0)),
            pl.BlockSpec((PAIR_LANES, H1F), lambda i: (0, 0)),
            pl.BlockSpec((1, H1F), lambda i: (0, 0)),
            pl.BlockSpec((NG, GRP * H1, GRP * H2), lambda i: (0, 0, 0)),
            pl.BlockSpec((1, H2F), lambda i: (0, 0)),
            pl.BlockSpec((H2F, M), lambda i: (0, 0)),
            pl.BlockSpec((1, M), lambda i: (0, 0)),
            pl.BlockSpec((F, H1), lambda i: (0, 0)),
            pl.BlockSpec((1, H1), lambda i: (0, 0)),
            pl.BlockSpec((H1, M), lambda i: (0, 0)),
            pl.BlockSpec((1, M), lambda i: (0, 0)),
        ],
        out_specs=[
            pl.BlockSpec((T, M), lambda i: (i, 0)),
            pl.BlockSpec((1, M), lambda i: (0, 0)),
            pl.BlockSpec((1, PAIR_LANES), lambda i: (0, 0)),
            pl.BlockSpec((1, M), lambda i: (0, 0)),
        ],
        out_shape=[
            jax.ShapeDtypeStruct((B, M), jnp.float32),
            jax.ShapeDtypeStruct((1, M), jnp.float32),
            jax.ShapeDtypeStruct((1, PAIR_LANES), jnp.int32),
            jax.ShapeDtypeStruct((1, M), jnp.float32),
        ],
        scratch_shapes=[pltpu.VMEM((F, H1F), jnp.float32)],
    )(x, W_int, W1row, b1f, W2bd, b2f, W3col, b3r, Wc1T, bc1r, Wc2T, bc2r)
    selected_pairs = jnp.stack([sel[0, :M], sel[0, M:]], axis=1)
    return (feat, vals[0], cmean[0], selected_pairs)


# speedup vs baseline: 1.8362x; 1.0150x over previous
import functools

import jax
import jax.numpy as jnp
from jax.experimental import pallas as pl
from jax.experimental.pallas import tpu as pltpu

F = 100
M = 20
H1 = 64
H2 = 32
PAIR_LANES = 2 * M
GRP = 4
NG = M // GRP
H1F = M * H1
H2F = M * H2


def _topk_body(Wint_ref, W1row_ref, vals_ref, sel_ref, G_ref):
    W = Wint_ref[...]
    row = jax.lax.broadcasted_iota(jnp.int32, (F, F), 0)
    col = jax.lax.broadcasted_iota(jnp.int32, (F, F), 1)
    flat = row * F + col
    s = jnp.where(col > row, jax.nn.sigmoid(W), -1.0)
    lane = jax.lax.broadcasted_iota(jnp.int32, (1, M), 1)
    vals = jnp.zeros((1, M), jnp.float32)
    idxs = jnp.zeros((1, M), jnp.int32)
    for k in range(M):
        m = jnp.max(s)
        cand = jnp.where(s == m, flat, jnp.int32(2**31 - 1))
        idx = jnp.min(cand)
        vals = jnp.where(lane == k, m, vals)
        idxs = jnp.where(lane == k, idx, idxs)
        s = jnp.where(flat == idx, -1.0, s)
    sel_i = idxs // F
    sel_j = idxs - sel_i * F
    sel = jnp.concatenate([sel_i, sel_j], axis=1)
    vals_ref[...] = vals
    sel_ref[...] = sel
    frow = jax.lax.broadcasted_iota(jnp.int32, (F, PAIR_LANES), 0)
    S = (frow == jnp.broadcast_to(sel, (F, PAIR_LANES))).astype(jnp.float32)
    G_ref[...] = jnp.dot(S, W1row_ref[...], preferred_element_type=jnp.float32)


def _main_body(x_ref, G_ref, b1f_ref, W2bd_ref, b2f_ref, W3col_ref,
               b3_ref, Wc1T_ref, bc1_ref, Wc2T_ref, bc2_ref,
               feat_ref, csum_ref, *, num_tiles, inv_b):
    pid = pl.program_id(0)
    xt = x_ref[...]
    hc = jnp.maximum(
        jnp.dot(xt, Wc1T_ref[...], preferred_element_type=jnp.float32)
        + bc1_ref[...], 0.0)
    cw = jax.nn.sigmoid(
        jnp.dot(hc, Wc2T_ref[...], preferred_element_type=jnp.float32)
        + bc2_ref[...])

    h1 = jnp.maximum(
        jnp.dot(xt, G_ref[...], preferred_element_type=jnp.float32)
        + b1f_ref[...], 0.0)
    h2g = []
    for g in range(NG):
        hg = jnp.dot(h1[:, g * GRP * H1:(g + 1) * GRP * H1], W2bd_ref[g],
                     preferred_element_type=jnp.float32)
        h2g.append(jnp.maximum(
            hg + b2f_ref[:, g * GRP * H2:(g + 1) * GRP * H2], 0.0))
    h2 = jnp.concatenate(h2g, axis=1)
    o = jnp.dot(h2, W3col_ref[...], preferred_element_type=jnp.float32)
    feat_ref[...] = (o + b3_ref[...]) * cw

    @pl.when(pid == 0)
    def _init():
        csum_ref[...] = jnp.zeros_like(csum_ref)

    csum_ref[...] += jnp.sum(cw, axis=0, keepdims=True)

    @pl.when(pid == num_tiles - 1)
    def _fin():
        csum_ref[...] *= inv_b


@jax.jit
def kernel(x, W_int, W1, b1, W2, b2, W3, b3, Wc1, bc1, Wc2, bc2):
    B = x.shape[0]
    T = 1024
    n = B // T
    W1row = jnp.zeros((PAIR_LANES, H1F), jnp.float32)
    for m in range(M):
        W1row = W1row.at[m, m * H1:(m + 1) * H1].set(W1[m, :, 0])
        W1row = W1row.at[M + m, m * H1:(m + 1) * H1].set(W1[m, :, 1])
    b1f = b1.reshape(1, H1F)
    W2T = jnp.transpose(W2, (0, 2, 1))
    W2bd = jnp.zeros((NG, GRP * H1, GRP * H2), jnp.float32)
    for g in range(NG):
        for ml in range(GRP):
            W2bd = W2bd.at[g, ml * H1:(ml + 1) * H1,
                           ml * H2:(ml + 1) * H2].set(W2T[g * GRP + ml])
    b2f = b2.reshape(1, H2F)
    W3col = jnp.zeros((H2F, M), jnp.float32)
    for m in range(M):
        W3col = W3col.at[m * H2:(m + 1) * H2, m].set(W3[m, 0, :])
    b3r = jnp.reshape(b3, (1, M))
    Wc1T = Wc1.T
    Wc2T = Wc2.T
    bc1r = bc1.reshape(1, H1)
    bc2r = bc2.reshape(1, M)

    vals, sel, G = pl.pallas_call(
        _topk_body,
        out_shape=[
            jax.ShapeDtypeStruct((1, M), jnp.float32),
            jax.ShapeDtypeStruct((1, PAIR_LANES), jnp.int32),
            jax.ShapeDtypeStruct((F, H1F), jnp.float32),
        ],
    )(W_int, W1row)

    feat, cmean = pl.pallas_call(
        functools.partial(_main_body, num_tiles=n, inv_b=1.0 / B),
        grid=(n,),
        in_specs=[
            pl.BlockSpec((T, F), lambda i: (i, 0)),
            pl.BlockSpec((F, H1F), lambda i: (0, 0)),
            pl.BlockSpec((1, H1F), lambda i: (0, 0)),
            pl.BlockSpec((NG, GRP * H1, GRP * H2), lambda i: (0, 0, 0)),
            pl.BlockSpec((1, H2F), lambda i: (0, 0)),
            pl.BlockSpec((H2F, M), lambda i: (0, 0)),
            pl.BlockSpec((1, M), lambda i: (0, 0)),
            pl.BlockSpec((F, H1), lambda i: (0, 0)),
            pl.BlockSpec((1, H1), lambda i: (0, 0)),
            pl.BlockSpec((H1, M), lambda i: (0, 0)),
            pl.BlockSpec((1, M), lambda i: (0, 0)),
        ],
        out_specs=[
            pl.BlockSpec((T, M), lambda i: (i, 0)),
            pl.BlockSpec((1, M), lambda i: (0, 0)),
        ],
        out_shape=[
            jax.ShapeDtypeStruct((B, M), jnp.float32),
            jax.ShapeDtypeStruct((1, M), jnp.float32),
        ],
    )(x, G, b1f, W2bd, b2f, W3col, b3r, Wc1T, bc1r, Wc2T, bc2r)
    selected_pairs = jnp.stack([sel[0, :M], sel[0, M:]], axis=1)
    return (feat, vals[0], cmean[0], selected_pairs)


# speedup vs baseline: 2.8903x; 1.5741x over previous
import functools

import jax
import jax.numpy as jnp
from jax.experimental import pallas as pl
from jax.experimental.pallas import tpu as pltpu

F = 100
M = 20
H1 = 64
H2 = 32
PAIR_LANES = 2 * M
GRP = 4
NG = M // GRP
H1F = M * H1
H2F = M * H2


def _topk_body(Wint_ref, W1row_ref, vals_ref, sel_ref, G_ref):
    W = Wint_ref[...]
    row = jax.lax.broadcasted_iota(jnp.int32, (F, F), 0)
    col = jax.lax.broadcasted_iota(jnp.int32, (F, F), 1)
    flat = row * F + col
    s = jnp.where(col > row, jax.nn.sigmoid(W), -1.0)
    lane = jax.lax.broadcasted_iota(jnp.int32, (1, M), 1)
    vals = jnp.zeros((1, M), jnp.float32)
    idxs = jnp.zeros((1, M), jnp.int32)
    for k in range(M):
        m = jnp.max(s)
        cand = jnp.where(s == m, flat, jnp.int32(2**31 - 1))
        idx = jnp.min(cand)
        vals = jnp.where(lane == k, m, vals)
        idxs = jnp.where(lane == k, idx, idxs)
        s = jnp.where(flat == idx, -1.0, s)
    sel_i = idxs // F
    sel_j = idxs - sel_i * F
    sel = jnp.concatenate([sel_i, sel_j], axis=1)
    vals_ref[...] = vals
    sel_ref[...] = sel
    frow = jax.lax.broadcasted_iota(jnp.int32, (F, PAIR_LANES), 0)
    S = (frow == jnp.broadcast_to(sel, (F, PAIR_LANES))).astype(jnp.float32)
    G_ref[...] = jnp.dot(S, W1row_ref[...], preferred_element_type=jnp.float32)


def _main_body(x_ref, G_ref, b1f_ref, W2bd_ref, b2f_ref, W3col_ref,
               b3_ref, Wc1T_ref, bc1_ref, Wc2T_ref, bc2_ref,
               feat_ref, csum_ref, *, num_tiles, inv_b):
    pid = pl.program_id(0)
    xt = x_ref[...]
    hc = jnp.maximum(
        jnp.dot(xt, Wc1T_ref[...], preferred_element_type=jnp.float32)
        + bc1_ref[...], 0.0)
    cw = jax.nn.sigmoid(
        jnp.dot(hc, Wc2T_ref[...], preferred_element_type=jnp.float32)
        + bc2_ref[...])

    h1 = jnp.maximum(
        jnp.dot(xt, G_ref[...], preferred_element_type=jnp.float32)
        + b1f_ref[...], 0.0)
    h2g = []
    for g in range(NG):
        hg = jnp.dot(h1[:, g * GRP * H1:(g + 1) * GRP * H1], W2bd_ref[g],
                     preferred_element_type=jnp.float32)
        h2g.append(jnp.maximum(
            hg + b2f_ref[:, g * GRP * H2:(g + 1) * GRP * H2], 0.0))
    h2 = jnp.concatenate(h2g, axis=1)
    o = jnp.dot(h2, W3col_ref[...], preferred_element_type=jnp.float32)
    feat_ref[...] = (o + b3_ref[...]) * cw

    @pl.when(pid == 0)
    def _init():
        csum_ref[...] = jnp.zeros_like(csum_ref)

    csum_ref[...] += jnp.sum(cw, axis=0, keepdims=True)

    @pl.when(pid == num_tiles - 1)
    def _fin():
        csum_ref[...] *= inv_b


@jax.jit
def kernel(x, W_int, W1, b1, W2, b2, W3, b3, Wc1, bc1, Wc2, bc2):
    B = x.shape[0]
    T = 1024
    n = B // T
    eyeM = jnp.eye(M, dtype=jnp.float32)
    W1a = (W1[:, None, :, 0] * eyeM[:, :, None]).reshape(M, H1F)
    W1b = (W1[:, None, :, 1] * eyeM[:, :, None]).reshape(M, H1F)
    W1row = jnp.concatenate([W1a, W1b], axis=0)
    b1f = b1.reshape(1, H1F)
    W2T = jnp.transpose(W2, (0, 2, 1)).reshape(NG, GRP, H1, H2)
    eyeG = jnp.eye(GRP, dtype=jnp.float32)
    W2bd = (W2T[:, :, :, None, :] *
            eyeG[None, :, None, :, None]).reshape(NG, GRP * H1, GRP * H2)
    b2f = b2.reshape(1, H2F)
    W3col = (W3[:, 0, :, None] * eyeM[:, None, :]).reshape(H2F, M)
    b3r = jnp.reshape(b3, (1, M))
    Wc1T = Wc1.T
    Wc2T = Wc2.T
    bc1r = bc1.reshape(1, H1)
    bc2r = bc2.reshape(1, M)

    vals, sel, G = pl.pallas_call(
        _topk_body,
        out_shape=[
            jax.ShapeDtypeStruct((1, M), jnp.float32),
            jax.ShapeDtypeStruct((1, PAIR_LANES), jnp.int32),
            jax.ShapeDtypeStruct((F, H1F), jnp.float32),
        ],
    )(W_int, W1row)

    feat, cmean = pl.pallas_call(
        functools.partial(_main_body, num_tiles=n, inv_b=1.0 / B),
        grid=(n,),
        in_specs=[
            pl.BlockSpec((T, F), lambda i: (i, 0)),
            pl.BlockSpec((F, H1F), lambda i: (0, 0)),
            pl.BlockSpec((1, H1F), lambda i: (0, 0)),
            pl.BlockSpec((NG, GRP * H1, GRP * H2), lambda i: (0, 0, 0)),
            pl.BlockSpec((1, H2F), lambda i: (0, 0)),
            pl.BlockSpec((H2F, M), lambda i: (0, 0)),
            pl.BlockSpec((1, M), lambda i: (0, 0)),
            pl.BlockSpec((F, H1), lambda i: (0, 0)),
            pl.BlockSpec((1, H1), lambda i: (0, 0)),
            pl.BlockSpec((H1, M), lambda i: (0, 0)),
            pl.BlockSpec((1, M), lambda i: (0, 0)),
        ],
        out_specs=[
            pl.BlockSpec((T, M), lambda i: (i, 0)),
            pl.BlockSpec((1, M), lambda i: (0, 0)),
        ],
        out_shape=[
            jax.ShapeDtypeStruct((B, M), jnp.float32),
            jax.ShapeDtypeStruct((1, M), jnp.float32),
        ],
    )(x, G, b1f, W2bd, b2f, W3col, b3r, Wc1T, bc1r, Wc2T, bc2r)
    selected_pairs = jnp.stack([sel[0, :M], sel[0, M:]], axis=1)
    return (feat, vals[0], cmean[0], selected_pairs)


# speedup vs baseline: 2.9730x; 1.0286x over previous
import functools

import jax
import jax.numpy as jnp
from jax.experimental import pallas as pl
from jax.experimental.pallas import tpu as pltpu

F = 100
M = 20
H1 = 64
H2 = 32
PAIR_LANES = 2 * M
GRP = 4
NG = M // GRP
H1F = M * H1
H2F = M * H2


def _topk_body(Wint_ref, W1row_ref, vals_ref, sel_ref, G_ref):
    W = Wint_ref[...]
    row = jax.lax.broadcasted_iota(jnp.int32, (F, F), 0)
    col = jax.lax.broadcasted_iota(jnp.int32, (F, F), 1)
    flat = row * F + col
    s = jnp.where(col > row, jax.nn.sigmoid(W), -1.0)
    lane = jax.lax.broadcasted_iota(jnp.int32, (1, M), 1)
    vals = jnp.zeros((1, M), jnp.float32)
    idxs = jnp.zeros((1, M), jnp.int32)
    for k in range(M):
        m = jnp.max(s)
        cand = jnp.where(s == m, flat, jnp.int32(2**31 - 1))
        idx = jnp.min(cand)
        vals = jnp.where(lane == k, m, vals)
        idxs = jnp.where(lane == k, idx, idxs)
        s = jnp.where(flat == idx, -1.0, s)
    sel_i = idxs // F
    sel_j = idxs - sel_i * F
    sel = jnp.concatenate([sel_i, sel_j], axis=1)
    vals_ref[...] = vals
    sel_ref[...] = sel
    frow = jax.lax.broadcasted_iota(jnp.int32, (F, PAIR_LANES), 0)
    S = (frow == jnp.broadcast_to(sel, (F, PAIR_LANES))).astype(jnp.float32)
    G_ref[...] = jnp.dot(S, W1row_ref[...], preferred_element_type=jnp.float32)


def _main_body(x_ref, G_ref, b1f_ref, W2bd_ref, b2f_ref, W3col_ref,
               b3_ref, Wc1T_ref, bc1_ref, Wc2T_ref, bc2_ref,
               feat_ref, csum_ref, *, num_tiles, inv_b):
    pid = pl.program_id(0)
    xt = x_ref[...]
    hc = jnp.maximum(
        jnp.dot(xt, Wc1T_ref[...], preferred_element_type=jnp.float32)
        + bc1_ref[...], 0.0)
    cw = jax.nn.sigmoid(
        jnp.dot(hc, Wc2T_ref[...], preferred_element_type=jnp.float32)
        + bc2_ref[...])

    h1 = jnp.maximum(
        jnp.dot(xt, G_ref[...], preferred_element_type=jnp.float32)
        + b1f_ref[...], 0.0)
    h2g = []
    for g in range(NG):
        hg = jnp.dot(h1[:, g * GRP * H1:(g + 1) * GRP * H1], W2bd_ref[g],
                     preferred_element_type=jnp.float32)
        h2g.append(jnp.maximum(
            hg + b2f_ref[:, g * GRP * H2:(g + 1) * GRP * H2], 0.0))
    h2 = jnp.concatenate(h2g, axis=1)
    o = jnp.dot(h2, W3col_ref[...], preferred_element_type=jnp.float32)
    feat_ref[...] = (o + b3_ref[...]) * cw

    @pl.when(pid == 0)
    def _init():
        csum_ref[...] = jnp.zeros_like(csum_ref)

    csum_ref[...] += jnp.sum(cw, axis=0, keepdims=True)

    @pl.when(pid == num_tiles - 1)
    def _fin():
        csum_ref[...] *= inv_b


@jax.jit
def kernel(x, W_int, W1, b1, W2, b2, W3, b3, Wc1, bc1, Wc2, bc2):
    B = x.shape[0]
    T = 2048
    n = B // T
    eyeM = jnp.eye(M, dtype=jnp.float32)
    W1a = (W1[:, None, :, 0] * eyeM[:, :, None]).reshape(M, H1F)
    W1b = (W1[:, None, :, 1] * eyeM[:, :, None]).reshape(M, H1F)
    W1row = jnp.concatenate([W1a, W1b], axis=0)
    b1f = b1.reshape(1, H1F)
    W2T = jnp.transpose(W2, (0, 2, 1)).reshape(NG, GRP, H1, H2)
    eyeG = jnp.eye(GRP, dtype=jnp.float32)
    W2bd = (W2T[:, :, :, None, :] *
            eyeG[None, :, None, :, None]).reshape(NG, GRP * H1, GRP * H2)
    b2f = b2.reshape(1, H2F)
    W3col = (W3[:, 0, :, None] * eyeM[:, None, :]).reshape(H2F, M)
    b3r = jnp.reshape(b3, (1, M))
    Wc1T = Wc1.T
    Wc2T = Wc2.T
    bc1r = bc1.reshape(1, H1)
    bc2r = bc2.reshape(1, M)

    vals, sel, G = pl.pallas_call(
        _topk_body,
        out_shape=[
            jax.ShapeDtypeStruct((1, M), jnp.float32),
            jax.ShapeDtypeStruct((1, PAIR_LANES), jnp.int32),
            jax.ShapeDtypeStruct((F, H1F), jnp.float32),
        ],
    )(W_int, W1row)

    feat, cmean = pl.pallas_call(
        functools.partial(_main_body, num_tiles=n, inv_b=1.0 / B),
        grid=(n,),
        in_specs=[
            pl.BlockSpec((T, F), lambda i: (i, 0)),
            pl.BlockSpec((F, H1F), lambda i: (0, 0)),
            pl.BlockSpec((1, H1F), lambda i: (0, 0)),
            pl.BlockSpec((NG, GRP * H1, GRP * H2), lambda i: (0, 0, 0)),
            pl.BlockSpec((1, H2F), lambda i: (0, 0)),
            pl.BlockSpec((H2F, M), lambda i: (0, 0)),
            pl.BlockSpec((1, M), lambda i: (0, 0)),
            pl.BlockSpec((F, H1), lambda i: (0, 0)),
            pl.BlockSpec((1, H1), lambda i: (0, 0)),
            pl.BlockSpec((H1, M), lambda i: (0, 0)),
            pl.BlockSpec((1, M), lambda i: (0, 0)),
        ],
        out_specs=[
            pl.BlockSpec((T, M), lambda i: (i, 0)),
            pl.BlockSpec((1, M), lambda i: (0, 0)),
        ],
        out_shape=[
            jax.ShapeDtypeStruct((B, M), jnp.float32),
            jax.ShapeDtypeStruct((1, M), jnp.float32),
        ],
    )(x, G, b1f, W2bd, b2f, W3col, b3r, Wc1T, bc1r, Wc2T, bc2r)
    selected_pairs = jnp.stack([sel[0, :M], sel[0, M:]], axis=1)
    return (feat, vals[0], cmean[0], selected_pairs)


# speedup vs baseline: 2.9953x; 1.0075x over previous
import functools

import jax
import jax.numpy as jnp
from jax.experimental import pallas as pl
from jax.experimental.pallas import tpu as pltpu

F = 100
M = 20
H1 = 64
H2 = 32
PAIR_LANES = 2 * M
GRP = 4
NG = M // GRP
H1F = M * H1
H2F = M * H2


def _topk_body(Wint_ref, W1row_ref, vals_ref, sel_ref, G_ref):
    W = Wint_ref[...]
    row = jax.lax.broadcasted_iota(jnp.int32, (F, F), 0)
    col = jax.lax.broadcasted_iota(jnp.int32, (F, F), 1)
    flat = row * F + col
    s = jnp.where(col > row, jax.nn.sigmoid(W), -1.0)
    lane = jax.lax.broadcasted_iota(jnp.int32, (1, M), 1)
    vals = jnp.zeros((1, M), jnp.float32)
    idxs = jnp.zeros((1, M), jnp.int32)
    for k in range(M):
        m = jnp.max(s)
        cand = jnp.where(s == m, flat, jnp.int32(2**31 - 1))
        idx = jnp.min(cand)
        vals = jnp.where(lane == k, m, vals)
        idxs = jnp.where(lane == k, idx, idxs)
        s = jnp.where(flat == idx, -1.0, s)
    sel_i = idxs // F
    sel_j = idxs - sel_i * F
    sel = jnp.concatenate([sel_i, sel_j], axis=1)
    vals_ref[...] = vals
    sel_ref[...] = sel
    frow = jax.lax.broadcasted_iota(jnp.int32, (F, PAIR_LANES), 0)
    S = (frow == jnp.broadcast_to(sel, (F, PAIR_LANES))).astype(jnp.float32)
    G_ref[...] = jnp.dot(S, W1row_ref[...], preferred_element_type=jnp.float32)


def _main_body(x_ref, G_ref, b1f_ref, W2bd_ref, b2f_ref, W3col_ref,
               b3_ref, Wc1T_ref, bc1_ref, Wc2T_ref, bc2_ref,
               feat_ref, csum_ref, *, num_tiles, inv_b):
    pid = pl.program_id(0)
    xt = x_ref[...]
    hc = jnp.maximum(
        jnp.dot(xt, Wc1T_ref[...], preferred_element_type=jnp.float32)
        + bc1_ref[...], 0.0)
    cw = jax.nn.sigmoid(
        jnp.dot(hc, Wc2T_ref[...], preferred_element_type=jnp.float32)
        + bc2_ref[...])

    h1 = jnp.maximum(
        jnp.dot(xt, G_ref[...], preferred_element_type=jnp.float32)
        + b1f_ref[...], 0.0)
    h1b = h1.astype(jnp.bfloat16)
    h2g = []
    for g in range(NG):
        hg = jnp.dot(h1b[:, g * GRP * H1:(g + 1) * GRP * H1], W2bd_ref[g],
                     preferred_element_type=jnp.float32)
        h2g.append(jnp.maximum(
            hg + b2f_ref[:, g * GRP * H2:(g + 1) * GRP * H2], 0.0))
    h2 = jnp.concatenate(h2g, axis=1)
    o = jnp.dot(h2.astype(jnp.bfloat16), W3col_ref[...],
                preferred_element_type=jnp.float32)
    feat_ref[...] = (o + b3_ref[...]) * cw

    @pl.when(pid == 0)
    def _init():
        csum_ref[...] = jnp.zeros_like(csum_ref)

    csum_ref[...] += jnp.sum(cw, axis=0, keepdims=True)

    @pl.when(pid == num_tiles - 1)
    def _fin():
        csum_ref[...] *= inv_b


@jax.jit
def kernel(x, W_int, W1, b1, W2, b2, W3, b3, Wc1, bc1, Wc2, bc2):
    B = x.shape[0]
    T = 2048
    n = B // T
    eyeM = jnp.eye(M, dtype=jnp.float32)
    W1a = (W1[:, None, :, 0] * eyeM[:, :, None]).reshape(M, H1F)
    W1b = (W1[:, None, :, 1] * eyeM[:, :, None]).reshape(M, H1F)
    W1row = jnp.concatenate([W1a, W1b], axis=0)
    b1f = b1.reshape(1, H1F)
    W2T = jnp.transpose(W2, (0, 2, 1)).reshape(NG, GRP, H1, H2)
    eyeG = jnp.eye(GRP, dtype=jnp.float32)
    W2bd = (W2T[:, :, :, None, :] *
            eyeG[None, :, None, :, None]).reshape(
                NG, GRP * H1, GRP * H2).astype(jnp.bfloat16)
    b2f = b2.reshape(1, H2F)
    W3col = (W3[:, 0, :, None] * eyeM[:, None, :]).reshape(
        H2F, M).astype(jnp.bfloat16)
    b3r = jnp.reshape(b3, (1, M))
    Wc1T = Wc1.T
    Wc2T = Wc2.T
    bc1r = bc1.reshape(1, H1)
    bc2r = bc2.reshape(1, M)

    vals, sel, G = pl.pallas_call(
        _topk_body,
        out_shape=[
            jax.ShapeDtypeStruct((1, M), jnp.float32),
            jax.ShapeDtypeStruct((1, PAIR_LANES), jnp.int32),
            jax.ShapeDtypeStruct((F, H1F), jnp.float32),
        ],
    )(W_int, W1row)

    feat, cmean = pl.pallas_call(
        functools.partial(_main_body, num_tiles=n, inv_b=1.0 / B),
        grid=(n,),
        in_specs=[
            pl.BlockSpec((T, F), lambda i: (i, 0)),
            pl.BlockSpec((F, H1F), lambda i: (0, 0)),
            pl.BlockSpec((1, H1F), lambda i: (0, 0)),
            pl.BlockSpec((NG, GRP * H1, GRP * H2), lambda i: (0, 0, 0)),
            pl.BlockSpec((1, H2F), lambda i: (0, 0)),
            pl.BlockSpec((H2F, M), lambda i: (0, 0)),
            pl.BlockSpec((1, M), lambda i: (0, 0)),
            pl.BlockSpec((F, H1), lambda i: (0, 0)),
            pl.BlockSpec((1, H1), lambda i: (0, 0)),
            pl.BlockSpec((H1, M), lambda i: (0, 0)),
            pl.BlockSpec((1, M), lambda i: (0, 0)),
        ],
        out_specs=[
            pl.BlockSpec((T, M), lambda i: (i, 0)),
            pl.BlockSpec((1, M), lambda i: (0, 0)),
        ],
        out_shape=[
            jax.ShapeDtypeStruct((B, M), jnp.float32),
            jax.ShapeDtypeStruct((1, M), jnp.float32),
        ],
    )(x, G, b1f, W2bd, b2f, W3col, b3r, Wc1T, bc1r, Wc2T, bc2r)
    selected_pairs = jnp.stack([sel[0, :M], sel[0, M:]], axis=1)
    return (feat, vals[0], cmean[0], selected_pairs)
